# baseline (speedup 1.0000x reference)
import functools

import jax
import jax.numpy as jnp
from jax import lax
from jax.experimental import pallas as pl
from jax.experimental.pallas import tpu as pltpu
from jax.experimental.pallas import tpu_sc as plsc


def _sigmoid(v):
    return 1.0 / (1.0 + jnp.exp2(v * -1.4426950408889634))


_NC = 2
_NS = 16
_NW = _NC * _NS
_XW = 16


def _prep_call(h, x16, A, B):
    N, H = h.shape

    def body(h_ref, x_ref, a_ref, b_ref, p_ref, q_ref):
        hb = h_ref[...]
        xb = x_ref[...]
        p_ref[:, :H] = jnp.dot(hb, a_ref[...], preferred_element_type=jnp.float32)
        p_ref[:, H:] = -xb
        q_ref[:, :H] = jnp.dot(hb, b_ref[...], preferred_element_type=jnp.float32)
        q_ref[:, H:] = xb

    BN = 2000
    return pl.pallas_call(
        body,
        grid=(N // BN,),
        in_specs=[
            pl.BlockSpec((BN, H), lambda i: (i, 0)),
            pl.BlockSpec((BN, _XW), lambda i: (i, 0)),
            pl.BlockSpec((H, H), lambda i: (0, 0)),
            pl.BlockSpec((H, H), lambda i: (0, 0)),
        ],
        out_specs=[
            pl.BlockSpec((BN, H + _XW), lambda i: (i, 0)),
            pl.BlockSpec((BN, H + _XW), lambda i: (i, 0)),
        ],
        out_shape=[
            jax.ShapeDtypeStruct((N, H + _XW), jnp.float32),
            jax.ShapeDtypeStruct((N, H + _XW), jnp.float32),
        ],
    )(h, x16, A, B)


def _gather_call(Pp, Qp, src2, dst2):
    N, W = Pp.shape
    ROWS, CH = src2.shape
    RPW = ROWS // _NW
    NB = 4
    ITER = RPW // NB
    TAIL = RPW - NB * ITER
    mesh = plsc.VectorSubcoreMesh(core_axis_name="c", subcore_axis_name="s")

    vm = pltpu.VMEM
    scratch = []
    for _ in range(NB):
        scratch += [vm((CH,), jnp.int32), vm((CH,), jnp.int32),
                    vm((CH, W), jnp.float32), pltpu.SemaphoreType.DMA]

    @functools.partial(
        pl.kernel,
        mesh=mesh,
        compiler_params=pltpu.CompilerParams(use_tc_tiling_on_sc=False),
        out_type=jax.ShapeDtypeStruct((ROWS, CH, W), jnp.float32),
        scratch_types=scratch,
    )
    def k(p_hbm, q_hbm, src_hbm, dst_hbm, pre_hbm, *scr):
        wid = lax.axis_index("s") * _NC + lax.axis_index("c")
        base = wid * RPW
        bufs = tuple(scr[4 * kk:4 * kk + 4] for kk in range(NB))

        def pstart(kk, r):
            is_, id_, buf, sg = bufs[kk]
            pltpu.sync_copy(src_hbm.at[r], is_)
            pltpu.sync_copy(dst_hbm.at[r], id_)
            pltpu.async_copy(p_hbm.at[is_], buf, sg)

        def qstart(kk):
            is_, id_, buf, sg = bufs[kk]
            pltpu.make_async_copy(p_hbm.at[is_], buf, sg).wait()
            pltpu.async_copy(q_hbm.at[id_], buf, sg, add=True)

        def fin(kk, r):
            is_, id_, buf, sg = bufs[kk]
            pltpu.make_async_copy(q_hbm.at[id_], buf, sg).wait()
            pltpu.sync_copy(buf, pre_hbm.at[r])

        for kk in range(NB):
            pstart(kk, base + kk)

        def body(i, carry):
            r4 = base + NB * i
            for kk in range(NB):
                qstart(kk)
            for kk in range(NB):
                fin(kk, r4 + kk)

                @pl.when(NB * i + kk + NB < RPW)
                def _():
                    pstart(kk, r4 + kk + NB)
            return carry

        lax.fori_loop(0, ITER, body, 0)
        for kk in range(TAIL):
            qstart(kk)
        for kk in range(TAIL):
            fin(kk, base + NB * ITER + kk)

    return k(Pp, Qp, src2, dst2)


def _edge_call(pre144, edge_attr, C, wd, be1, We2, be2, Wx1, bx1, wx2, bx2):
    E, W = pre144.shape
    H = W - _XW
    BE = 2000

    def body(pre_ref, ea_ref, c_ref, wd_ref, be1_ref, we2_ref,
             be2_ref, wx1_ref, bx1_ref, wx2_ref, bx2_ref, m_ref, cm_ref):
        pd = pre_ref[...]
        d = pd[:, H:]
        dsq = d * d
        ones_col = jnp.ones((_XW, 1), jnp.float32)
        s = jnp.dot(dsq, ones_col, preferred_element_type=jnp.float32) + 1e-9
        r = lax.rsqrt(s)
        dist = s * r
        pre = (pd[:, :H]
               + jnp.dot(ea_ref[...], c_ref[...], preferred_element_type=jnp.float32)
               + dist * wd_ref[...] + be1_ref[...])
        m1 = pre * _sigmoid(pre)
        m = jnp.dot(m1, we2_ref[...], preferred_element_type=jnp.float32) + be2_ref[...]
        m_ref[...] = m
        g = jnp.dot(m, wx1_ref[...], preferred_element_type=jnp.float32) + bx1_ref[...]
        g = g * _sigmoid(g)
        gate = jnp.dot(g, wx2_ref[...], preferred_element_type=jnp.float32) + bx2_ref[...]
        cm_ref[...] = d * (gate * r)

    full = lambda shape: pl.BlockSpec(shape, lambda i: (0,) * len(shape))
    return pl.pallas_call(
        body,
        grid=(E // BE,),
        in_specs=[
            pl.BlockSpec((BE, W), lambda i: (i, 0)),
            pl.BlockSpec((BE, 16), lambda i: (i, 0)),
            full((16, H)), full((1, H)), full((1, H)), full((H, H)),
            full((1, H)), full((H, H)), full((1, H)), full((H, 1)),
            full((1, 1)),
        ],
        out_specs=[
            pl.BlockSpec((BE, H), lambda i: (i, 0)),
            pl.BlockSpec((BE, _XW), lambda i: (i, 0)),
        ],
        out_shape=[
            jax.ShapeDtypeStruct((E, H), jnp.float32),
            jax.ShapeDtypeStruct((E, _XW), jnp.float32),
        ],
    )(pre144, edge_attr, C, wd, be1, We2, be2, Wx1, bx1, wx2, bx2)


_CHS = 40
_NBS = 6


def _scatter_call(m3, cm3, dst2, N):
    ROWS, CHS, H = m3.shape
    RPW = ROWS // _NW
    ITER = RPW // _NBS
    TAIL = RPW - _NBS * ITER
    RN = N // _NS
    ZR = 25
    mesh = plsc.VectorSubcoreMesh(core_axis_name="c", subcore_axis_name="s")

    vm = pltpu.VMEM
    scratch = []
    for _ in range(_NBS):
        scratch += [vm((CHS,), jnp.int32), vm((CHS, H), jnp.float32),
                    vm((CHS, _XW), jnp.float32),
                    pltpu.SemaphoreType.DMA, pltpu.SemaphoreType.DMA]
    scratch += [
        vm((ZR, H), jnp.float32),
        vm((ZR, _XW), jnp.float32),
        pltpu.VMEM_SHARED((N, H), jnp.float32),
        pltpu.VMEM_SHARED((N, _XW), jnp.float32),
    ]

    @functools.partial(
        pl.kernel,
        mesh=mesh,
        compiler_params=pltpu.CompilerParams(use_tc_tiling_on_sc=False),
        out_type=(
            jax.ShapeDtypeStruct((_NC, N, H), jnp.float32),
            jax.ShapeDtypeStruct((_NC, N, _XW), jnp.float32),
        ),
        scratch_types=scratch,
    )
    def k(m_hbm, cm_hbm, dst_hbm, agg_hbm, dx_hbm, *scr):
        c = lax.axis_index("c")
        s = lax.axis_index("s")
        wid = s * _NC + c
        bufs = tuple(scr[5 * kk:5 * kk + 5] for kk in range(_NBS))
        z_m, z_c, acc_a, acc_x = scr[5 * _NBS:5 * _NBS + 4]

        def zbody(i, carry):
            for j in range(H // 16):
                z_m[i, pl.ds(j * 16, 16)] = jnp.zeros((16,), jnp.float32)
            z_c[i, :] = jnp.zeros((16,), jnp.float32)
            return carry

        lax.fori_loop(0, ZR, zbody, 0)
        tbase = s * RN
        for kk in range(RN // ZR):
            pltpu.sync_copy(z_m, acc_a.at[pl.ds(tbase + kk * ZR, ZR)])
            pltpu.sync_copy(z_c, acc_x.at[pl.ds(tbase + kk * ZR, ZR)])
        plsc.subcore_barrier()

        base = wid * RPW

        def load(kk, r):
            ix, bm, bc, sl, ss = bufs[kk]
            pltpu.sync_copy(dst_hbm.at[r], ix)
            pltpu.async_copy(m_hbm.at[r], bm, sl)
            pltpu.async_copy(cm_hbm.at[r], bc, sl)

        def scatter(kk, r):
            ix, bm, bc, sl, ss = bufs[kk]
            pltpu.make_async_copy(m_hbm.at[r], bm, sl).wait()
            pltpu.make_async_copy(cm_hbm.at[r], bc, sl).wait()
            pltpu.async_copy(bm, acc_a.at[ix], ss, add=True)
            pltpu.async_copy(bc, acc_x.at[ix], ss, add=True)

        def wait_scatter(kk):
            ix, bm, bc, sl, ss = bufs[kk]
            pltpu.make_async_copy(bm, acc_a.at[ix], ss).wait()
            pltpu.make_async_copy(bc, acc_x.at[ix], ss).wait()

        for kk in range(_NBS):
            load(kk, base + kk)

        def row_body(i, carry):
            r0 = base + _NBS * i
            for kk in range(_NBS):
                scatter(kk, r0 + kk)
            for kk in range(_NBS):
                wait_scatter(kk)

                @pl.when(_NBS * i + kk + _NBS < RPW)
                def _():
                    load(kk, r0 + kk + _NBS)
            return carry

        lax.fori_loop(0, ITER, row_body, 0)
        for kk in range(TAIL):
            scatter(kk, base + _NBS * ITER + kk)
        for kk in range(TAIL):
            wait_scatter(kk)
        plsc.subcore_barrier()

        for kk in range(RN // ZR):
            off = tbase + kk * ZR
            pltpu.sync_copy(acc_a.at[pl.ds(off, ZR)], z_m)
            pltpu.sync_copy(z_m, agg_hbm.at[c, pl.ds(off, ZR)])
            pltpu.sync_copy(acc_x.at[pl.ds(off, ZR)], z_c)
            pltpu.sync_copy(z_c, dx_hbm.at[c, pl.ds(off, ZR)])

    return k(m3, cm3, dst2)


def _node_call(h, x16, aggp, aggp2, dxp, dxp2, Wh1a, Wh1b, bh1, Wh2, bh2,
               ln_g, ln_b):
    N, H = h.shape
    BN = 2000

    def body(h_ref, x_ref, agg_ref, agg2_ref, dx_ref, dx2_ref, wa_ref, wb_ref,
             bh1_ref, wh2_ref, bh2_ref, g_ref, b_ref, ho_ref, xo_ref):
        hb = h_ref[...]
        agg = (agg_ref[0] + agg_ref[1]) + (agg2_ref[0] + agg2_ref[1])
        t = (jnp.dot(hb, wa_ref[...], preferred_element_type=jnp.float32)
             + jnp.dot(agg, wb_ref[...], preferred_element_type=jnp.float32)
             + bh1_ref[...])
        t = t * _sigmoid(t)
        dh = jnp.dot(t, wh2_ref[...], preferred_element_type=jnp.float32) + bh2_ref[...]
        pre = hb + dh
        mu = jnp.mean(pre, axis=1, keepdims=True)
        ctr = pre - mu
        var = jnp.mean(ctr * ctr, axis=1, keepdims=True)
        ho_ref[...] = ctr / jnp.sqrt(var + 1e-5) * g_ref[...] + b_ref[...]
        xo_ref[...] = x_ref[...] + (dx_ref[0] + dx_ref[1]) + (dx2_ref[0] + dx2_ref[1])

    full = lambda shape: pl.BlockSpec(shape, lambda i: (0,) * len(shape))
    return pl.pallas_call(
        body,
        grid=(N // BN,),
        in_specs=[
            pl.BlockSpec((BN, H), lambda i: (i, 0)),
            pl.BlockSpec((BN, _XW), lambda i: (i, 0)),
            pl.BlockSpec((_NC, BN, H), lambda i: (0, i, 0)),
            pl.BlockSpec((_NC, BN, H), lambda i: (0, i, 0)),
            pl.BlockSpec((_NC, BN, _XW), lambda i: (0, i, 0)),
            pl.BlockSpec((_NC, BN, _XW), lambda i: (0, i, 0)),
            full((H, H)), full((H, H)), full((1, H)), full((H, H)),
            full((1, H)), full((1, H)), full((1, H)),
        ],
        out_specs=[
            pl.BlockSpec((BN, H), lambda i: (i, 0)),
            pl.BlockSpec((BN, _XW), lambda i: (i, 0)),
        ],
        out_shape=[
            jax.ShapeDtypeStruct((N, H), jnp.float32),
            jax.ShapeDtypeStruct((N, _XW), jnp.float32),
        ],
    )(h, x16, aggp, aggp2, dxp, dxp2, Wh1a, Wh1b, bh1, Wh2, bh2, ln_g, ln_b)


def kernel(h, x, edge_index, edge_attr, We1, be1, We2, be2,
           Wh1, bh1, Wh2, bh2, Wx1, bx1, Wx2, bx2, ln_g, ln_b):
    N, H = h.shape
    E = edge_index.shape[1]
    ED = edge_attr.shape[1]
    EH = E // 2
    CHG = 40

    x16 = jnp.pad(x, ((0, 0), (0, _XW - x.shape[1])))
    ei = edge_index.astype(jnp.int32)

    A = We1[:H]
    B = We1[H:2 * H]
    C = We1[2 * H:2 * H + ED]
    wd = We1[2 * H + ED:2 * H + ED + 1]

    Pp, Qp = _prep_call(h, x16, A, B)

    parts = []
    for half in range(2):
        sl = slice(half * EH, (half + 1) * EH)
        src2 = ei[0, sl].reshape(EH // CHG, CHG)
        dst2 = ei[1, sl].reshape(EH // CHG, CHG)
        pre3 = _gather_call(Pp, Qp, src2, dst2)
        m, cm = _edge_call(
            pre3.reshape(EH, H + _XW), edge_attr[sl],
            C, wd, be1.reshape(1, H), We2, be2.reshape(1, H),
            Wx1, bx1.reshape(1, H), Wx2, bx2.reshape(1, 1))
        dst2s = ei[1, sl].reshape(EH // _CHS, _CHS)
        aggp, dxp = _scatter_call(
            m.reshape(EH // _CHS, _CHS, H), cm.reshape(EH // _CHS, _CHS, _XW),
            dst2s, N)
        parts.append((aggp, dxp))

    h_out, x16o = _node_call(
        h, x16, parts[0][0], parts[1][0], parts[0][1], parts[1][1],
        Wh1[:H], Wh1[H:], bh1.reshape(1, H),
        Wh2, bh2.reshape(1, H), ln_g.reshape(1, H), ln_b.reshape(1, H))
    return h_out, x16o[:, :x.shape[1]]
